# baseline (speedup 1.0000x reference)
import functools

import jax
import jax.numpy as jnp
from jax import lax
from jax.experimental import pallas as pl
from jax.experimental.pallas import tpu as pltpu
from jax.experimental.pallas import tpu_sc as plsc

T = 2048
D = 1024
DP = D // 2
E = 8
CAP = 640
S = E * CAP
DFF = 3072
DH = DFF // 2
NC, NS = 2, 16
NW = NC * NS
RPT = S // NW
TPT = T // NW
EPS = (2 * T) // NS


def _route_body(x_ref, wg_ref, slot0_ref, slot1_ref, keep0_ref, keep1_ref,
                w0p_ref, w1p_ref):
    x = x_ref[...]
    wg = wg_ref[...]
    logits = jnp.dot(x, wg, preferred_element_type=jnp.float32)
    lt = logits.T
    row = lax.broadcasted_iota(jnp.int32, (E, T), 0)
    m0 = jnp.max(lt, axis=0, keepdims=True)
    i0 = jnp.min(jnp.where(lt == m0, row, E), axis=0, keepdims=True)
    sel0 = row == i0
    masked = jnp.where(sel0, -jnp.inf, lt)
    m1 = jnp.max(masked, axis=0, keepdims=True)
    i1 = jnp.min(jnp.where(masked == m1, row, E), axis=0, keepdims=True)
    sel1 = row == i1
    ez = jnp.exp(lt - m0)
    ssum = jnp.sum(ez, axis=0, keepdims=True)
    w0 = 1.0 / ssum
    w1 = jnp.exp(m1 - m0) / ssum
    cnt = sel0.astype(jnp.float32) + sel1.astype(jnp.float32)
    c = cnt
    sh = 1
    while sh < T:
        c = c + jnp.concatenate(
            [jnp.zeros((E, sh), jnp.float32), c[:, :T - sh]], axis=1)
        sh *= 2
    cex = c - cnt
    pos0 = jnp.sum(jnp.where(sel0, cex, 0.0), axis=0, keepdims=True)
    pos1 = jnp.sum(jnp.where(sel1, cex, 0.0), axis=0, keepdims=True)
    keep0 = (pos0 < CAP).astype(jnp.int32)
    keep1 = (pos1 < CAP).astype(jnp.int32)
    p0 = jnp.minimum(pos0, CAP - 1.0).astype(jnp.int32)
    p1 = jnp.minimum(pos1, CAP - 1.0).astype(jnp.int32)
    slot0_ref[...] = i0 * CAP + p0
    slot1_ref[...] = i1 * CAP + p1
    keep0_ref[...] = keep0
    keep1_ref[...] = keep1
    cw0 = (w0 * keep0.astype(jnp.float32)).reshape(T, 1)
    cw1 = (w1 * keep1.astype(jnp.float32)).reshape(T, 1)
    w0p_ref[...] = jnp.broadcast_to(cw0, (T, 16))
    w1p_ref[...] = jnp.broadcast_to(cw1, (T, 16))


def _route(x2, w_gate):
    return pl.pallas_call(
        _route_body,
        out_shape=[
            jax.ShapeDtypeStruct((1, T), jnp.int32),
            jax.ShapeDtypeStruct((1, T), jnp.int32),
            jax.ShapeDtypeStruct((1, T), jnp.int32),
            jax.ShapeDtypeStruct((1, T), jnp.int32),
            jax.ShapeDtypeStruct((T, 16), jnp.float32),
            jax.ShapeDtypeStruct((T, 16), jnp.float32),
        ],
    )(x2, w_gate)


def _dispatch_body(slot0_hbm, slot1_hbm, keep0_hbm, keep1_hbm, xp_hbm,
                   out_hbm, es_v, ek_v, ia_v, ta_v, table, idx_v, rows_v,
                   sem):
    c = lax.axis_index("c")
    s = lax.axis_index("s")
    lanes = lax.iota(jnp.int32, 16)

    def do_half(slot_hbm, keep_hbm, toff):
        for g in range(2):
            tb = toff + g * 128
            pltpu.sync_copy(slot_hbm.at[0, pl.ds(tb, 128)], es_v)
            pltpu.sync_copy(keep_hbm.at[0, pl.ds(tb, 128)], ek_v)
            for ch in range(8):
                sl = pl.ds(ch * 16, 16)
                ia_v[sl] = jnp.where(ek_v[sl] != 0, es_v[sl], S)
                ta_v[sl] = tb + ch * 16 + lanes
            pltpu.sync_copy(ta_v, table.at[ia_v])

    @pl.when(s < NS // 2)
    def _():
        do_half(slot0_hbm, keep0_hbm, s * EPS)

    @pl.when(s >= NS // 2)
    def _():
        do_half(slot1_hbm, keep1_hbm, (s - NS // 2) * EPS)

    plsc.subcore_barrier()

    base = (s * NC + c) * RPT
    pltpu.sync_copy(table.at[pl.ds(base, RPT)], idx_v)
    for ch in range(RPT // 16):
        sl = pl.ds(ch * 16, 16)
        idx_v[sl] = jnp.clip(idx_v[sl], 0, T - 1)
    h = RPT // 2
    cp0 = pltpu.async_copy(xp_hbm.at[idx_v.at[pl.ds(0, h)]],
                           rows_v.at[pl.ds(0, h)], sem)
    cp1 = pltpu.async_copy(xp_hbm.at[idx_v.at[pl.ds(h, h)]],
                           rows_v.at[pl.ds(h, h)], sem)
    cp0.wait()
    cp1.wait()
    pltpu.sync_copy(rows_v, out_hbm.at[pl.ds(base, RPT)])


def _mlp_body(xb_ref, fc_ref, pj_ref, out_ref):
    a = xb_ref[...]
    w1 = fc_ref[0].astype(jnp.bfloat16)
    h = jnp.dot(a, w1, preferred_element_type=jnp.float32)
    u = h[:, :DH]
    g = h[:, DH:]
    hh = (u * lax.logistic(u) * g).astype(jnp.bfloat16)
    w2 = pj_ref[0].astype(jnp.bfloat16)
    out_ref[...] = jnp.dot(hh, w2, preferred_element_type=jnp.float32)


def _mlp(exp_xb, c_fc, c_proj):
    return pl.pallas_call(
        _mlp_body,
        grid=(E,),
        in_specs=[
            pl.BlockSpec((CAP, D), lambda e: (e, 0)),
            pl.BlockSpec((1, D, DFF), lambda e: (e, 0, 0)),
            pl.BlockSpec((1, DH, D), lambda e: (e, 0, 0)),
        ],
        out_specs=pl.BlockSpec((CAP, D), lambda e: (e, 0)),
        out_shape=jax.ShapeDtypeStruct((S, D), jnp.float32),
    )(exp_xb, c_fc, c_proj)


CCH = TPT // 2


def _combine_body(slot0_hbm, slot1_hbm, w0p_hbm, w1p_hbm, eo_hbm, out_hbm,
                  i0_v, i1_v, w0_v, w1_v, x_v, y_v, z_v, gsem, wsem):
    wid = lax.axis_index("s") * NC + lax.axis_index("c")
    tb = wid * TPT
    pltpu.sync_copy(slot0_hbm.at[0, pl.ds(tb, TPT)], i0_v)
    pltpu.sync_copy(slot1_hbm.at[0, pl.ds(tb, TPT)], i1_v)
    pltpu.sync_copy(w0p_hbm.at[pl.ds(tb, TPT)], w0_v)
    pltpu.sync_copy(w1p_hbm.at[pl.ds(tb, TPT)], w1_v)

    def weighted_sum(acc_v, oth_v, toff):
        def tok_body(t, carry):
            wv0 = w0_v[toff + t, :]
            wv1 = w1_v[toff + t, :]
            for ch in range(D // 16):
                sl = pl.ds(ch * 16, 16)
                acc_v[t, sl] = acc_v[t, sl] * wv0 + oth_v[t, sl] * wv1
            return carry

        lax.fori_loop(0, CCH, tok_body, 0)

    ga0 = pltpu.async_copy(eo_hbm.at[i0_v.at[pl.ds(0, CCH)]], x_v, gsem)
    ga1 = pltpu.async_copy(eo_hbm.at[i1_v.at[pl.ds(0, CCH)]], y_v, gsem)
    ga0.wait()
    ga1.wait()
    gb0 = pltpu.async_copy(eo_hbm.at[i0_v.at[pl.ds(CCH, CCH)]], z_v, gsem)
    weighted_sum(x_v, y_v, 0)
    wa = pltpu.async_copy(x_v, out_hbm.at[pl.ds(tb, CCH)], wsem)
    gb0.wait()
    gb1 = pltpu.async_copy(eo_hbm.at[i1_v.at[pl.ds(CCH, CCH)]], y_v, gsem)
    gb1.wait()
    weighted_sum(z_v, y_v, CCH)
    wa.wait()
    pltpu.sync_copy(z_v, out_hbm.at[pl.ds(tb + CCH, CCH)])


@functools.cache
def _sc_kernels():
    mesh = plsc.VectorSubcoreMesh(core_axis_name="c", subcore_axis_name="s",
                                  num_cores=NC, num_subcores=NS)
    dispatch = pl.kernel(
        _dispatch_body,
        out_type=jax.ShapeDtypeStruct((S, DP), jnp.float32),
        mesh=mesh,
        scratch_types=[
            pltpu.VMEM((128,), jnp.int32),
            pltpu.VMEM((128,), jnp.int32),
            pltpu.VMEM((128,), jnp.int32),
            pltpu.VMEM((128,), jnp.int32),
            pltpu.VMEM_SHARED((S + 8,), jnp.int32),
            pltpu.VMEM((RPT,), jnp.int32),
            pltpu.VMEM((RPT, DP), jnp.float32),
            pltpu.SemaphoreType.DMA,
        ],
    )
    combine = pl.kernel(
        _combine_body,
        out_type=jax.ShapeDtypeStruct((T, D), jnp.float32),
        mesh=mesh,
        scratch_types=[
            pltpu.VMEM((TPT,), jnp.int32),
            pltpu.VMEM((TPT,), jnp.int32),
            pltpu.VMEM((TPT, 16), jnp.float32),
            pltpu.VMEM((TPT, 16), jnp.float32),
            pltpu.VMEM((CCH, D), jnp.float32),
            pltpu.VMEM((CCH, D), jnp.float32),
            pltpu.VMEM((CCH, D), jnp.float32),
            pltpu.SemaphoreType.DMA,
            pltpu.SemaphoreType.DMA,
        ],
    )
    return dispatch, combine


def kernel(x, w_gate, c_fc, c_proj):
    x2 = x.reshape(T, D)
    slot0, slot1, keep0, keep1, w0p, w1p = _route(x2, w_gate)
    dispatch, combine = _sc_kernels()
    xp = lax.bitcast_convert_type(
        x2.astype(jnp.bfloat16).reshape(T, DP, 2), jnp.float32)
    exp_xp = dispatch(slot0, slot1, keep0, keep1, xp)
    exp_xb = lax.bitcast_convert_type(
        exp_xp, jnp.bfloat16).reshape(S, D)
    exp_out = _mlp(exp_xb, c_fc, c_proj)
    out = combine(slot0, slot1, w0p, w1p, exp_out)
    return out.reshape(1, T, D)


# speedup vs baseline: 2.0570x; 2.0570x over previous
import functools

import jax
import jax.numpy as jnp
from jax import lax
from jax.experimental import pallas as pl
from jax.experimental.pallas import tpu as pltpu
from jax.experimental.pallas import tpu_sc as plsc

T = 2048
D = 1024
DP = D // 2
E = 8
CAP = 640
S = E * CAP
DFF = 3072
DH = DFF // 2
NC, NS = 2, 16
NW = NC * NS
RPT = S // NW
TPT = T // NW
EPS = (2 * T) // NS


def _route_body(x_ref, wg_ref, slot0_ref, slot1_ref, keep0_ref, keep1_ref,
                w0p_ref, w1p_ref):
    x = x_ref[...]
    wg = wg_ref[...]
    logits = jnp.dot(x, wg, preferred_element_type=jnp.float32)
    lt = logits.T
    row = lax.broadcasted_iota(jnp.int32, (E, T), 0)
    m0 = jnp.max(lt, axis=0, keepdims=True)
    i0 = jnp.min(jnp.where(lt == m0, row, E), axis=0, keepdims=True)
    sel0 = row == i0
    masked = jnp.where(sel0, -jnp.inf, lt)
    m1 = jnp.max(masked, axis=0, keepdims=True)
    i1 = jnp.min(jnp.where(masked == m1, row, E), axis=0, keepdims=True)
    sel1 = row == i1
    ez = jnp.exp(lt - m0)
    ssum = jnp.sum(ez, axis=0, keepdims=True)
    w0 = 1.0 / ssum
    w1 = jnp.exp(m1 - m0) / ssum
    cnt = sel0.astype(jnp.float32) + sel1.astype(jnp.float32)
    c = cnt
    sh = 1
    while sh < T:
        c = c + jnp.concatenate(
            [jnp.zeros((E, sh), jnp.float32), c[:, :T - sh]], axis=1)
        sh *= 2
    cex = c - cnt
    pos0 = jnp.sum(jnp.where(sel0, cex, 0.0), axis=0, keepdims=True)
    pos1 = jnp.sum(jnp.where(sel1, cex, 0.0), axis=0, keepdims=True)
    keep0 = (pos0 < CAP).astype(jnp.int32)
    keep1 = (pos1 < CAP).astype(jnp.int32)
    p0 = jnp.minimum(pos0, CAP - 1.0).astype(jnp.int32)
    p1 = jnp.minimum(pos1, CAP - 1.0).astype(jnp.int32)
    slot0_ref[...] = i0 * CAP + p0
    slot1_ref[...] = i1 * CAP + p1
    keep0_ref[...] = keep0
    keep1_ref[...] = keep1
    cw0 = (w0 * keep0.astype(jnp.float32)).reshape(T, 1)
    cw1 = (w1 * keep1.astype(jnp.float32)).reshape(T, 1)
    w0p_ref[...] = jnp.broadcast_to(cw0, (T, 16))
    w1p_ref[...] = jnp.broadcast_to(cw1, (T, 16))


def _route(x2, w_gate):
    return pl.pallas_call(
        _route_body,
        out_shape=[
            jax.ShapeDtypeStruct((1, T), jnp.int32),
            jax.ShapeDtypeStruct((1, T), jnp.int32),
            jax.ShapeDtypeStruct((1, T), jnp.int32),
            jax.ShapeDtypeStruct((1, T), jnp.int32),
            jax.ShapeDtypeStruct((T, 16), jnp.float32),
            jax.ShapeDtypeStruct((T, 16), jnp.float32),
        ],
    )(x2, w_gate)


def _dispatch_body(slot0_hbm, slot1_hbm, keep0_hbm, keep1_hbm, xp_hbm,
                   out_hbm, es_v, ek_v, ia_v, ta_v, table, idx_v, rows_v,
                   sem):
    c = lax.axis_index("c")
    s = lax.axis_index("s")
    lanes = lax.iota(jnp.int32, 16)

    def do_half(slot_hbm, keep_hbm, toff):
        for g in range(2):
            tb = toff + g * 128
            pltpu.sync_copy(slot_hbm.at[0, pl.ds(tb, 128)], es_v)
            pltpu.sync_copy(keep_hbm.at[0, pl.ds(tb, 128)], ek_v)
            for ch in range(8):
                sl = pl.ds(ch * 16, 16)
                ia_v[sl] = jnp.where(ek_v[sl] != 0, es_v[sl], S)
                ta_v[sl] = tb + ch * 16 + lanes
            pltpu.sync_copy(ta_v, table.at[ia_v])

    @pl.when(s < NS // 2)
    def _():
        do_half(slot0_hbm, keep0_hbm, s * EPS)

    @pl.when(s >= NS // 2)
    def _():
        do_half(slot1_hbm, keep1_hbm, (s - NS // 2) * EPS)

    plsc.subcore_barrier()

    base = (s * NC + c) * RPT
    pltpu.sync_copy(table.at[pl.ds(base, RPT)], idx_v)
    for ch in range(RPT // 16):
        sl = pl.ds(ch * 16, 16)
        idx_v[sl] = jnp.clip(idx_v[sl], 0, T - 1)
    h = RPT // 2
    for ch in range(2):
        pltpu.async_copy(xp_hbm.at[idx_v.at[pl.ds(ch * h, h)]],
                         rows_v, sem).wait()
        pltpu.sync_copy(rows_v, out_hbm.at[pl.ds(base + ch * h, h)])


def _mlp_body(xb_ref, fc_ref, pj_ref, out_ref):
    a = xb_ref[...].astype(jnp.bfloat16)
    w1 = fc_ref[0].astype(jnp.bfloat16)
    h = jnp.dot(a, w1, preferred_element_type=jnp.float32)
    u = h[:, :DH]
    g = h[:, DH:]
    hh = (u * lax.logistic(u) * g).astype(jnp.bfloat16)
    w2 = pj_ref[0].astype(jnp.bfloat16)
    out_ref[...] = jnp.dot(hh, w2, preferred_element_type=jnp.float32)


def _mlp(exp_xb, c_fc, c_proj):
    return pl.pallas_call(
        _mlp_body,
        grid=(E,),
        in_specs=[
            pl.BlockSpec((CAP, D), lambda e: (e, 0)),
            pl.BlockSpec((1, D, DFF), lambda e: (e, 0, 0)),
            pl.BlockSpec((1, DH, D), lambda e: (e, 0, 0)),
        ],
        out_specs=pl.BlockSpec((CAP, D), lambda e: (e, 0)),
        out_shape=jax.ShapeDtypeStruct((S, D), jnp.float32),
    )(exp_xb, c_fc, c_proj)


CCH = TPT // 2


def _combine_body(slot0_hbm, slot1_hbm, w0p_hbm, w1p_hbm, eo_hbm, out_hbm,
                  i0_v, i1_v, w0_v, w1_v, x_v, y_v, z_v, gsem, wsem):
    wid = lax.axis_index("s") * NC + lax.axis_index("c")
    tb = wid * TPT
    pltpu.sync_copy(slot0_hbm.at[0, pl.ds(tb, TPT)], i0_v)
    pltpu.sync_copy(slot1_hbm.at[0, pl.ds(tb, TPT)], i1_v)
    pltpu.sync_copy(w0p_hbm.at[pl.ds(tb, TPT)], w0_v)
    pltpu.sync_copy(w1p_hbm.at[pl.ds(tb, TPT)], w1_v)

    def weighted_sum(acc_v, oth_v, toff):
        def tok_body(t, carry):
            wv0 = w0_v[toff + t, :]
            wv1 = w1_v[toff + t, :]
            for ch in range(D // 16):
                sl = pl.ds(ch * 16, 16)
                acc_v[t, sl] = acc_v[t, sl] * wv0 + oth_v[t, sl] * wv1
            return carry

        lax.fori_loop(0, CCH, tok_body, 0)

    ga0 = pltpu.async_copy(eo_hbm.at[i0_v.at[pl.ds(0, CCH)]], x_v, gsem)
    ga1 = pltpu.async_copy(eo_hbm.at[i1_v.at[pl.ds(0, CCH)]], y_v, gsem)
    ga0.wait()
    ga1.wait()
    gb0 = pltpu.async_copy(eo_hbm.at[i0_v.at[pl.ds(CCH, CCH)]], z_v, gsem)
    weighted_sum(x_v, y_v, 0)
    wa = pltpu.async_copy(x_v, out_hbm.at[pl.ds(tb, CCH)], wsem)
    gb0.wait()
    gb1 = pltpu.async_copy(eo_hbm.at[i1_v.at[pl.ds(CCH, CCH)]], y_v, gsem)
    gb1.wait()
    weighted_sum(z_v, y_v, CCH)
    wa.wait()
    pltpu.sync_copy(z_v, out_hbm.at[pl.ds(tb + CCH, CCH)])


@functools.cache
def _sc_kernels():
    mesh = plsc.VectorSubcoreMesh(core_axis_name="c", subcore_axis_name="s",
                                  num_cores=NC, num_subcores=NS)
    dispatch = pl.kernel(
        _dispatch_body,
        out_type=jax.ShapeDtypeStruct((S, D), jnp.float32),
        mesh=mesh,
        scratch_types=[
            pltpu.VMEM((128,), jnp.int32),
            pltpu.VMEM((128,), jnp.int32),
            pltpu.VMEM((128,), jnp.int32),
            pltpu.VMEM((128,), jnp.int32),
            pltpu.VMEM_SHARED((S + 8,), jnp.int32),
            pltpu.VMEM((RPT,), jnp.int32),
            pltpu.VMEM((RPT // 2, D), jnp.float32),
            pltpu.SemaphoreType.DMA,
        ],
    )
    combine = pl.kernel(
        _combine_body,
        out_type=jax.ShapeDtypeStruct((T, D), jnp.float32),
        mesh=mesh,
        scratch_types=[
            pltpu.VMEM((TPT,), jnp.int32),
            pltpu.VMEM((TPT,), jnp.int32),
            pltpu.VMEM((TPT, 16), jnp.float32),
            pltpu.VMEM((TPT, 16), jnp.float32),
            pltpu.VMEM((CCH, D), jnp.float32),
            pltpu.VMEM((CCH, D), jnp.float32),
            pltpu.VMEM((CCH, D), jnp.float32),
            pltpu.SemaphoreType.DMA,
            pltpu.SemaphoreType.DMA,
        ],
    )
    return dispatch, combine


def kernel(x, w_gate, c_fc, c_proj):
    x2 = x.reshape(T, D)
    slot0, slot1, keep0, keep1, w0p, w1p = _route(x2, w_gate)
    dispatch, combine = _sc_kernels()
    exp_x = dispatch(slot0, slot1, keep0, keep1, x2)
    exp_out = _mlp(exp_x, c_fc, c_proj)
    out = combine(slot0, slot1, w0p, w1p, exp_out)
    return out.reshape(1, T, D)
